# Initial kernel scaffold; baseline (speedup 1.0000x reference)
import jax
import jax.numpy as jnp
from jax.experimental import pallas as pl


def kernel(x, edge_index, edge_attr, coords_init, fc1_W, fc1_b, k0_W, k0_b, k1_W, k1_b, k2_W, k2_b, root_W, node_b, c0_W, c0_b, c1_W, f2a_W, f2a_b, f2b_W, f2b_b):
    raise NotImplementedError("write your pallas kernel here")


# speedup vs baseline: 3.4497x; 3.4497x over previous
import functools

import numpy as np
import jax
import jax.numpy as jnp
from jax import lax
from jax.experimental import pallas as pl
from jax.experimental.pallas import tpu as pltpu
from jax.experimental.pallas import tpu_sc as plsc

N_NODES = 10000
E_EDGES = 160000
W = 16
GW = 32
SW = 48
CHUNK = 128
NWORK = 32
E_PAD = 163840
CH_TOT = E_PAD // CHUNK
CH_W = CH_TOT // NWORK
E_W = E_PAD // NWORK
NR_T = N_NODES // 16
EB = 1024

_f32 = jnp.float32


def _mesh():
    return plsc.VectorSubcoreMesh(core_axis_name="c", subcore_axis_name="s")


def _sc_gather(gtab, colc):
    @functools.partial(
        pl.kernel,
        mesh=_mesh(),
        compiler_params=pltpu.CompilerParams(use_tc_tiling_on_sc=False),
        out_type=jax.ShapeDtypeStruct((E_PAD, GW), _f32),
        scratch_types=[
            pltpu.VMEM((CH_W, CHUNK), jnp.int32),
            pltpu.VMEM((CHUNK, GW), _f32),
            pltpu.VMEM((CHUNK, GW), _f32),
            pltpu.SemaphoreType.DMA,
            pltpu.SemaphoreType.DMA,
        ],
    )
    def k(g_hbm, col_hbm, out_hbm, idx_v, buf0, buf1, sem0, sem1):
        wid = lax.axis_index("s") * 2 + lax.axis_index("c")
        pltpu.sync_copy(col_hbm.at[pl.ds(wid * CH_W, CH_W)], idx_v)
        bufs = (buf0, buf1)
        sems = (sem0, sem1)

        def body(i, carry):
            slot = lax.rem(i, 2)

            def run(b, s):
                pltpu.async_copy(g_hbm.at[idx_v.at[i]], b, s).wait()
                pltpu.sync_copy(b, out_hbm.at[pl.ds(wid * E_W + i * CHUNK, CHUNK)])

            @pl.when(slot == 0)
            def _():
                run(bufs[0], sems[0])

            @pl.when(slot == 1)
            def _():
                run(bufs[1], sems[1])

            return carry

        lax.fori_loop(0, CH_W, body, 0)

    return k(gtab, colc)


def _sc_scatter(vals, rowc):
    @functools.partial(
        pl.kernel,
        mesh=_mesh(),
        compiler_params=pltpu.CompilerParams(use_tc_tiling_on_sc=False),
        out_type=jax.ShapeDtypeStruct((2 * N_NODES, SW), _f32),
        scratch_types=[
            pltpu.VMEM((CH_W, CHUNK), jnp.int32),
            pltpu.VMEM((CHUNK, SW), _f32),
            pltpu.VMEM((NR_T, SW), _f32),
            pltpu.VMEM_SHARED((N_NODES, SW), _f32),
            pltpu.SemaphoreType.DMA,
        ],
    )
    def k(vals_hbm, row_hbm, out_hbm, idx_v, buf_v, stage_v, acc_sh, sem):
        c = lax.axis_index("c")
        s = lax.axis_index("s")
        wid = s * 2 + c

        zv = jnp.zeros((16,), _f32)

        def zbody(t, carry):
            stage_v[t // 3, pl.ds((t % 3) * 16, 16)] = zv
            return carry

        lax.fori_loop(0, NR_T * 3, zbody, 0)
        pltpu.sync_copy(stage_v, acc_sh.at[pl.ds(s * NR_T, NR_T)])
        plsc.subcore_barrier()

        pltpu.sync_copy(row_hbm.at[pl.ds(wid * CH_W, CH_W)], idx_v)

        def body(i, carry):
            pltpu.sync_copy(vals_hbm.at[pl.ds(wid * E_W + i * CHUNK, CHUNK)], buf_v)
            pltpu.sync_copy(buf_v, acc_sh.at[idx_v.at[i]], add=True)
            return carry

        lax.fori_loop(0, CH_W, body, 0)
        plsc.subcore_barrier()
        pltpu.sync_copy(
            acc_sh.at[pl.ds(s * NR_T, NR_T)],
            out_hbm.at[pl.ds(c * N_NODES + s * NR_T, NR_T)],
        )

    return k(vals, rowc)


def _tc_kmat(ea, k0_W, k0_b, k1_W, k1_b, k2_W, k2_b):
    def body(ea_ref, w0, b0, w1, b1, w2, b2, out_ref):
        a = jnp.maximum(jnp.dot(ea_ref[...], w0[...], preferred_element_type=_f32) + b0[...], 0.0)
        a = jnp.maximum(jnp.dot(a, w1[...], preferred_element_type=_f32) + b1[...], 0.0)
        out_ref[...] = jnp.dot(a, w2[...], preferred_element_type=_f32) + b2[...]

    full = lambda shape: pl.BlockSpec(shape, lambda i: (0, 0))
    return pl.pallas_call(
        body,
        grid=(E_PAD // EB,),
        in_specs=[
            pl.BlockSpec((EB, 16), lambda i: (i, 0)),
            full((16, 32)), full((1, 32)),
            full((32, 64)), full((1, 64)),
            full((64, 256)), full((1, 256)),
        ],
        out_specs=pl.BlockSpec((EB, 256), lambda i: (i, 0)),
        out_shape=jax.ShapeDtypeStruct((E_PAD, 256), _f32),
    )(ea, k0_W, k0_b, k1_W, k1_b, k2_W, k2_b)


def _tc_prep(x, coords, fc1_W, fc1_b):
    def body(x_ref, c_ref, w_ref, b_ref, g_ref):
        h = jnp.dot(x_ref[...], w_ref[...], preferred_element_type=_f32) + b_ref[...]
        z = jnp.zeros((N_NODES, GW - W - 3), _f32)
        g_ref[...] = jnp.concatenate([h, c_ref[...], z], axis=1)

    return pl.pallas_call(
        body,
        out_shape=jax.ShapeDtypeStruct((N_NODES, GW), _f32),
    )(x, coords, fc1_W, fc1_b)


def _tc_edge(kmat, hg, c0_W, c0_b, c1_W, tmat, smat):
    def body(k_ref, g_ref, c0w, c0b, c1w, t_ref, s_ref, out_ref):
        i = pl.program_id(0)
        kv = k_ref[...]
        gv = g_ref[...]
        h = gv[:, :W]
        cg = gv[:, W:W + 3]
        hrep = jnp.dot(h, t_ref[...], preferred_element_type=_f32)
        m = jnp.dot(kv * hrep, s_ref[...], preferred_element_type=_f32)
        t = jnp.maximum(jnp.dot(m, c0w[...], preferred_element_type=_f32) + c0b[...], 0.0)
        phi = jnp.dot(t, c1w[...], preferred_element_type=_f32)
        ones = jnp.ones((EB, 1), _f32)
        zz = jnp.zeros((EB, SW - W - 5), _f32)
        out = jnp.concatenate([m, cg * phi, phi, ones, zz], axis=1)
        eid = i * EB + lax.broadcasted_iota(jnp.int32, (EB, 1), 0)
        out_ref[...] = jnp.where(eid < E_EDGES, out, 0.0)

    full = lambda shape: pl.BlockSpec(shape, lambda i: (0, 0))
    return pl.pallas_call(
        body,
        grid=(E_PAD // EB,),
        in_specs=[
            pl.BlockSpec((EB, 256), lambda i: (i, 0)),
            pl.BlockSpec((EB, GW), lambda i: (i, 0)),
            full((16, 16)), full((1, 16)), full((16, 1)),
            full((16, 256)), full((256, 16)),
        ],
        out_specs=pl.BlockSpec((EB, SW), lambda i: (i, 0)),
        out_shape=jax.ShapeDtypeStruct((E_PAD, SW), _f32),
    )(kmat, hg, c0_W, c0_b, c1_W, tmat, smat)


def _tc_node(gtab, parts, root_W, node_b):
    def body(g_ref, p_ref, rw, nb, gout_ref, cnt_ref):
        P = p_ref[:N_NODES, :] + p_ref[N_NODES:, :]
        cnt = jnp.maximum(P[:, 20:21], 1.0)
        agg = P[:, :W] / cnt
        h = g_ref[:, :W]
        c3 = g_ref[:, W:W + 3]
        hn = jnp.maximum(jnp.dot(h, rw[...], preferred_element_type=_f32) + agg + nb[...], 0.0)
        c3n = c3 + (c3 * P[:, 19:20] - P[:, 16:19]) / cnt
        z = jnp.zeros((N_NODES, GW - W - 3), _f32)
        gout_ref[...] = jnp.concatenate([hn, c3n, z], axis=1)
        cnt_ref[...] = cnt

    return pl.pallas_call(
        body,
        out_shape=(
            jax.ShapeDtypeStruct((N_NODES, GW), _f32),
            jax.ShapeDtypeStruct((N_NODES, 1), _f32),
        ),
    )(gtab, parts, root_W, node_b)


def _tc_final(gtab, parts, cnt, root_W, node_b, f2a_W, f2a_b, f2b_W, f2b_b):
    def body(g_ref, p_ref, cnt_ref, rw, nb, aw, ab, bw, bb, out_ref, c_ref):
        P = p_ref[:N_NODES, :] + p_ref[N_NODES:, :]
        cnt = cnt_ref[...]
        agg = P[:, :W] / cnt
        h = g_ref[:, :W]
        c3 = g_ref[:, W:W + 3]
        hn = jnp.maximum(jnp.dot(h, rw[...], preferred_element_type=_f32) + agg + nb[...], 0.0)
        c_ref[...] = c3 + (c3 * P[:, 19:20] - P[:, 16:19]) / cnt
        a = jnp.maximum(jnp.dot(hn, aw[...], preferred_element_type=_f32) + ab[...], 0.0)
        out_ref[...] = jnp.dot(a, bw[...], preferred_element_type=_f32) + bb[...]

    return pl.pallas_call(
        body,
        out_shape=(
            jax.ShapeDtypeStruct((N_NODES, 1), _f32),
            jax.ShapeDtypeStruct((N_NODES, 3), _f32),
        ),
    )(gtab, parts, cnt, root_W, node_b, f2a_W, f2a_b, f2b_W, f2b_b)


_T_NP = np.tile(np.eye(16, dtype=np.float32), (1, 16))
_S_NP = np.kron(np.eye(16, dtype=np.float32), np.ones((16, 1), np.float32))


def kernel(x, edge_index, edge_attr, coords_init, fc1_W, fc1_b, k0_W, k0_b,
           k1_W, k1_b, k2_W, k2_b, root_W, node_b, c0_W, c0_b, c1_W,
           f2a_W, f2a_b, f2b_W, f2b_b):
    row = edge_index[0]
    col = edge_index[1]
    pad = E_PAD - E_EDGES
    colc = jnp.pad(col, (0, pad)).reshape(CH_TOT, CHUNK)
    rowc = jnp.pad(row, (0, pad)).reshape(CH_TOT, CHUNK)
    ea_p = jnp.pad(edge_attr, ((0, pad), (0, 0)))

    tmat = jnp.asarray(_T_NP)
    smat = jnp.asarray(_S_NP)
    kmat = _tc_kmat(ea_p, k0_W, k0_b.reshape(1, -1), k1_W, k1_b.reshape(1, -1),
                    k2_W, k2_b.reshape(1, -1))
    gtab = _tc_prep(x, coords_init, fc1_W, fc1_b.reshape(1, -1))

    hg = _sc_gather(gtab, colc)
    ev = _tc_edge(kmat, hg, c0_W, c0_b.reshape(1, -1), c1_W, tmat, smat)
    parts = _sc_scatter(ev, rowc)
    gtab, cnt = _tc_node(gtab, parts, root_W, node_b.reshape(1, -1))

    hg = _sc_gather(gtab, colc)
    ev = _tc_edge(kmat, hg, c0_W, c0_b.reshape(1, -1), c1_W, tmat, smat)
    parts = _sc_scatter(ev, rowc)
    out, coords = _tc_final(gtab, parts, cnt, root_W, node_b.reshape(1, -1),
                            f2a_W, f2a_b.reshape(1, -1), f2b_W, f2b_b.reshape(1, -1))
    return (out, coords)
